# baseline (speedup 1.0000x reference)
import functools
import numpy as np
import jax
import jax.numpy as jnp
from jax import lax
from jax.experimental import pallas as pl
from jax.experimental.pallas import tpu as pltpu, tpu_sc as plsc

_C = 192
_T = 128
_N = 1024
_SPLIT = 512
_BN = 128


_B = _SPLIT * _C
_NW = 32
_BPW = _B // _NW
_CHUNK = 128
_NCHUNK = _BPW // _CHUNK
_NBUF = 4
_NGRP = _NCHUNK // _NBUF


def _perm() -> np.ndarray:
    mixed = np.stack([np.arange(48, 96), np.arange(96, 144)]).T.reshape(-1)
    return np.concatenate([np.arange(0, 48), mixed, np.arange(144, 192)])


def _row_index() -> np.ndarray:
    r = np.arange(_B)
    return ((r // _C) * _C + _perm()[r % _C]).astype(np.int32)


_mesh = plsc.VectorSubcoreMesh(core_axis_name="c", subcore_axis_name="s")


@functools.partial(
    pl.kernel,
    mesh=_mesh,
    out_type=jax.ShapeDtypeStruct((_B, _T), jnp.float32),
    scratch_types=(
        [pltpu.VMEM((_CHUNK,), jnp.int32) for _ in range(_NBUF)]
        + [pltpu.VMEM((_CHUNK, _T), jnp.float32) for _ in range(_NBUF)]
        + [pltpu.SemaphoreType.DMA for _ in range(2 * _NBUF)]
    ),
)
def _sc_gather(table_hbm, idx_hbm, out_hbm,
               i0, i1, i2, i3, r0, r1, r2, r3,
               g0, g1, g2, g3, s0, s1, s2, s3):
    wid = lax.axis_index("s") * 2 + lax.axis_index("c")
    base = wid * _BPW
    idx_b = (i0, i1, i2, i3)
    row_b = (r0, r1, r2, r3)
    g_sem = (g0, g1, g2, g3)
    s_sem = (s0, s1, s2, s3)

    def gather_start(c, q):
        off = base + c * _CHUNK
        pltpu.sync_copy(idx_hbm.at[pl.ds(off, _CHUNK)], idx_b[q])
        pltpu.make_async_copy(table_hbm.at[idx_b[q]], row_b[q],
                              g_sem[q]).start()

    def gather_wait(q):
        pltpu.make_async_copy(table_hbm.at[idx_b[q]], row_b[q],
                              g_sem[q]).wait()

    def scatter_start(c, q):
        off = base + c * _CHUNK
        pltpu.make_async_copy(row_b[q], out_hbm.at[pl.ds(off, _CHUNK)],
                              s_sem[q]).start()

    def scatter_wait(c, q):
        off = base + c * _CHUNK
        pltpu.make_async_copy(row_b[q], out_hbm.at[pl.ds(off, _CHUNK)],
                              s_sem[q]).wait()

    for q in range(_NBUF):
        gather_start(q, q)
    for q in range(_NBUF):
        gather_wait(q)
        scatter_start(q, q)

    def body(p, _):
        for q in range(_NBUF):
            c = p * _NBUF + q
            scatter_wait(c - _NBUF, q)
            gather_start(c, q)
        for q in range(_NBUF):
            c = p * _NBUF + q
            gather_wait(q)
            scatter_start(c, q)
        return 0

    lax.fori_loop(1, _NGRP, body, 0, unroll=False)

    last = (_NGRP - 1) * _NBUF
    for q in range(_NBUF):
        scatter_wait(last + q, q)


def _permute_body(x_ref, o_ref):
    o_ref[:, 0:48, :] = x_ref[:, 0:48, :]
    o_ref[:, 48:144:2, :] = x_ref[:, 48:96, :]
    o_ref[:, 49:144:2, :] = x_ref[:, 96:144, :]
    o_ref[:, 144:192, :] = x_ref[:, 144:192, :]


def _tc_permute(xt):
    n = xt.shape[0]
    return pl.pallas_call(
        _permute_body,
        grid=(n // _BN,),
        in_specs=[pl.BlockSpec((_BN, _C, _T), lambda i: (i, 0, 0))],
        out_specs=pl.BlockSpec((_BN, _C, _T), lambda i: (i, 0, 0)),
        out_shape=jax.ShapeDtypeStruct((n, _C, _T), jnp.float32),
    )(xt)


def kernel(inputs):
    xt = jnp.swapaxes(inputs, 1, 2)
    idx = jnp.asarray(_row_index())
    sc_out = _sc_gather(xt[:_SPLIT].reshape(_B, _T), idx)
    tc_out = _tc_permute(xt[_SPLIT:])
    out = jnp.concatenate([sc_out.reshape(_SPLIT, _C, _T), tc_out], axis=0)
    return jnp.swapaxes(out, 1, 2)


# speedup vs baseline: 2.2954x; 2.2954x over previous
import functools
import numpy as np
import jax
import jax.numpy as jnp
from jax import lax
from jax.experimental import pallas as pl
from jax.experimental.pallas import tpu as pltpu, tpu_sc as plsc

_C = 192
_T = 128
_N = 1024
_B = _N * _C
_NW = 32
_BPW = _B // _NW
_CHUNK = 128
_NCHUNK = _BPW // _CHUNK
_NBUF = 6
_NGRP = _NCHUNK // _NBUF


def _perm() -> np.ndarray:
    mixed = np.stack([np.arange(48, 96), np.arange(96, 144)]).T.reshape(-1)
    return np.concatenate([np.arange(0, 48), mixed, np.arange(144, 192)])


def _row_index() -> np.ndarray:
    r = np.arange(_B)
    return ((r // _C) * _C + _perm()[r % _C]).astype(np.int32)


_mesh = plsc.VectorSubcoreMesh(core_axis_name="c", subcore_axis_name="s")


@functools.partial(
    pl.kernel,
    mesh=_mesh,
    out_type=jax.ShapeDtypeStruct((_B, _T), jnp.float32),
    scratch_types=(
        [pltpu.VMEM((_CHUNK,), jnp.int32) for _ in range(_NBUF)]
        + [pltpu.VMEM((_CHUNK, _T), jnp.float32) for _ in range(_NBUF)]
        + [pltpu.SemaphoreType.DMA for _ in range(2 * _NBUF)]
    ),
)
def _sc_gather(table_hbm, idx_hbm, out_hbm, *scr):
    wid = lax.axis_index("s") * 2 + lax.axis_index("c")
    base = wid * _BPW
    idx_b = scr[0:_NBUF]
    row_b = scr[_NBUF:2 * _NBUF]
    g_sem = scr[2 * _NBUF:3 * _NBUF]
    s_sem = scr[3 * _NBUF:4 * _NBUF]

    def gather_start(c, q):
        off = base + c * _CHUNK
        pltpu.sync_copy(idx_hbm.at[pl.ds(off, _CHUNK)], idx_b[q])
        pltpu.make_async_copy(table_hbm.at[idx_b[q]], row_b[q],
                              g_sem[q]).start()

    def gather_wait(q):
        pltpu.make_async_copy(table_hbm.at[idx_b[q]], row_b[q],
                              g_sem[q]).wait()

    def scatter_start(c, q):
        off = base + c * _CHUNK
        pltpu.make_async_copy(row_b[q], out_hbm.at[pl.ds(off, _CHUNK)],
                              s_sem[q]).start()

    def scatter_wait(c, q):
        off = base + c * _CHUNK
        pltpu.make_async_copy(row_b[q], out_hbm.at[pl.ds(off, _CHUNK)],
                              s_sem[q]).wait()

    for q in range(_NBUF):
        gather_start(q, q)
    for q in range(_NBUF):
        gather_wait(q)
        scatter_start(q, q)

    def body(p, _):
        for q in range(_NBUF):
            c = p * _NBUF + q
            scatter_wait(c - _NBUF, q)
            gather_start(c, q)
        for q in range(_NBUF):
            c = p * _NBUF + q
            gather_wait(q)
            scatter_start(c, q)
        return 0

    lax.fori_loop(1, _NGRP, body, 0, unroll=False)

    last = (_NGRP - 1) * _NBUF
    for q in range(_NBUF):
        scatter_wait(last + q, q)


def kernel(inputs):
    xt = jnp.swapaxes(inputs, 1, 2).reshape(_B, _T)
    idx = jnp.asarray(_row_index())
    out = _sc_gather(xt, idx)
    return jnp.swapaxes(out.reshape(_N, _C, _T), 1, 2)


# speedup vs baseline: 2.4243x; 1.0562x over previous
import functools
import numpy as np
import jax
import jax.numpy as jnp
from jax import lax
from jax.experimental import pallas as pl
from jax.experimental.pallas import tpu as pltpu, tpu_sc as plsc

_C = 192
_T = 128
_N = 1024
_SPLIT = 384
_BN = 128

_BTOT = _N * _C
_B = _SPLIT * _C
_NW = 32
_BPW = _B // _NW
_CHUNK = 128
_NCHUNK = _BPW // _CHUNK
_NBUF = 3
_NGRP = _NCHUNK // _NBUF


def _perm() -> np.ndarray:
    mixed = np.stack([np.arange(48, 96), np.arange(96, 144)]).T.reshape(-1)
    return np.concatenate([np.arange(0, 48), mixed, np.arange(144, 192)])


def _row_index() -> np.ndarray:
    r = np.arange(_B)
    return ((r // _C) * _C + _perm()[r % _C]).astype(np.int32)


_mesh = plsc.VectorSubcoreMesh(core_axis_name="c", subcore_axis_name="s")


@functools.partial(
    pl.kernel,
    mesh=_mesh,
    out_type=jax.ShapeDtypeStruct((_BTOT, _T), jnp.float32),
    scratch_types=(
        [pltpu.VMEM((_CHUNK,), jnp.int32) for _ in range(_NBUF)]
        + [pltpu.VMEM((_CHUNK, _T), jnp.float32) for _ in range(_NBUF)]
        + [pltpu.SemaphoreType.DMA for _ in range(2 * _NBUF)]
    ),
)
def _sc_gather(table_hbm, idx_hbm, out_hbm, *scr):
    wid = lax.axis_index("s") * 2 + lax.axis_index("c")
    base = wid * _BPW
    idx_b = scr[0:_NBUF]
    row_b = scr[_NBUF:2 * _NBUF]
    g_sem = scr[2 * _NBUF:3 * _NBUF]
    s_sem = scr[3 * _NBUF:4 * _NBUF]

    def gather_start(c, q):
        off = base + c * _CHUNK
        pltpu.sync_copy(idx_hbm.at[pl.ds(off, _CHUNK)], idx_b[q])
        pltpu.make_async_copy(table_hbm.at[idx_b[q]], row_b[q],
                              g_sem[q]).start()

    def gather_wait(q):
        pltpu.make_async_copy(table_hbm.at[idx_b[q]], row_b[q],
                              g_sem[q]).wait()

    def scatter_start(c, q):
        off = base + c * _CHUNK
        pltpu.make_async_copy(row_b[q], out_hbm.at[pl.ds(off, _CHUNK)],
                              s_sem[q]).start()

    def scatter_wait(c, q):
        off = base + c * _CHUNK
        pltpu.make_async_copy(row_b[q], out_hbm.at[pl.ds(off, _CHUNK)],
                              s_sem[q]).wait()

    for q in range(_NBUF):
        gather_start(q, q)
    for q in range(_NBUF):
        gather_wait(q)
        scatter_start(q, q)

    def body(p, _):
        for q in range(_NBUF):
            c = p * _NBUF + q
            scatter_wait(c - _NBUF, q)
            gather_start(c, q)
        for q in range(_NBUF):
            c = p * _NBUF + q
            gather_wait(q)
            scatter_start(c, q)
        return 0

    lax.fori_loop(1, _NGRP, body, 0, unroll=False)

    last = (_NGRP - 1) * _NBUF
    for q in range(_NBUF):
        scatter_wait(last + q, q)


def _permute_body(x_ref, _, o_ref):
    o_ref[:, 0:48, :] = x_ref[:, 0:48, :]
    o_ref[:, 48:144:2, :] = x_ref[:, 48:96, :]
    o_ref[:, 49:144:2, :] = x_ref[:, 96:144, :]
    o_ref[:, 144:192, :] = x_ref[:, 144:192, :]


def kernel(inputs):
    xt = jnp.swapaxes(inputs, 1, 2)
    idx = jnp.asarray(_row_index())
    sc_out = _sc_gather(xt.reshape(_BTOT, _T), idx).reshape(_N, _C, _T)
    off = _SPLIT // _BN
    out = pl.pallas_call(
        _permute_body,
        grid=((_N - _SPLIT) // _BN,),
        in_specs=[
            pl.BlockSpec((_BN, _C, _T), lambda i: (i + off, 0, 0)),
            pl.BlockSpec(memory_space=pl.ANY),
        ],
        out_specs=pl.BlockSpec((_BN, _C, _T), lambda i: (i + off, 0, 0)),
        out_shape=jax.ShapeDtypeStruct((_N, _C, _T), jnp.float32),
        input_output_aliases={1: 0},
    )(xt, sc_out)
    return jnp.swapaxes(out, 1, 2)
